# baseline (speedup 1.0000x reference)
import functools

import jax
import jax.numpy as jnp
from jax import lax
from jax.experimental import pallas as pl
from jax.experimental.pallas import tpu as pltpu
from jax.experimental.pallas import tpu_sc as plsc

_NUM_SC_CORES = 2
_NUM_SC_SUBCORES = 16
_NW = _NUM_SC_CORES * _NUM_SC_SUBCORES
_BN = 8192
_QN = _BN // 4


def _pack_pair(y):
    u = lax.bitcast_convert_type(y.astype(jnp.bfloat16), jnp.uint16)
    w = lax.convert_element_type(u, jnp.int32)
    return w[:, :32] | (w[:, 32:] << 16)


def _unpack_pair(w):
    lo = lax.convert_element_type(w & 0xFFFF, jnp.uint16)
    hi = lax.convert_element_type((w >> 16) & 0xFFFF, jnp.uint16)
    return jnp.concatenate(
        [lax.bitcast_convert_type(lo, jnp.bfloat16),
         lax.bitcast_convert_type(hi, jnp.bfloat16)], axis=1)


def _repack_body(t_ref, o_ref):
    x = t_ref[...]
    o_ref[:, 0:32] = _pack_pair(x[:, 0 * _QN:1 * _QN].T)
    o_ref[:, 32:64] = _pack_pair(x[:, 1 * _QN:2 * _QN].T)
    o_ref[:, 64:96] = _pack_pair(x[:, 2 * _QN:3 * _QN].T)
    o_ref[:, 96:128] = _pack_pair(x[:, 3 * _QN:4 * _QN].T)


def _make_repack(n_rows: int, emb: int):
    n_blocks = (n_rows + _BN - 1) // _BN

    def repack(table_t):
        return pl.pallas_call(
            _repack_body,
            grid=(n_blocks,),
            in_specs=[pl.BlockSpec((emb, _BN), lambda i: (0, i))],
            out_specs=pl.BlockSpec((_QN, 128), lambda i: (i, 0)),
            out_shape=jax.ShapeDtypeStruct((n_blocks * _QN, 128), jnp.int32),
        )(table_t)

    return repack


def _make_gather(batch: int):
    b_per_w = batch // _NW
    chunk = 128
    n_chunks = b_per_w // chunk
    mesh = plsc.VectorSubcoreMesh(
        core_axis_name="c", subcore_axis_name="s",
        num_cores=_NUM_SC_CORES, num_subcores=_NUM_SC_SUBCORES)

    @functools.partial(
        pl.kernel,
        mesh=mesh,
        out_type=[
            jax.ShapeDtypeStruct((batch, 128), jnp.int32),
            jax.ShapeDtypeStruct((batch, 128), jnp.int32),
        ],
        scratch_types=[
            pltpu.VMEM((b_per_w,), jnp.int32),
            pltpu.VMEM((b_per_w,), jnp.int32),
            pltpu.VMEM((chunk, 128), jnp.int32),
            pltpu.VMEM((chunk, 128), jnp.int32),
            pltpu.SemaphoreType.DMA,
            pltpu.SemaphoreType.DMA,
        ],
    )
    def gather_k(uids_hbm, iids_hbm, utab_hbm, itab_hbm, uout_hbm, iout_hbm,
                 uidx_v, iidx_v, buf0, buf1, sem0, sem1):
        wid = lax.axis_index("s") * _NUM_SC_CORES + lax.axis_index("c")
        base = wid * b_per_w
        pltpu.sync_copy(uids_hbm.at[pl.ds(base, b_per_w)], uidx_v)
        pltpu.sync_copy(iids_hbm.at[pl.ds(base, b_per_w)], iidx_v)

        plan = [(utab_hbm, uidx_v, uout_hbm, c) for c in range(n_chunks)]
        plan += [(itab_hbm, iidx_v, iout_hbm, c) for c in range(n_chunks)]
        bufs = [buf0, buf1]
        sems = [sem0, sem1]
        copies = []
        for k, (tab, idx, _, c) in enumerate(plan):
            if k >= 2:
                copies[k - 2].wait()
                _, _, out, pc = plan[k - 2]
                pltpu.sync_copy(bufs[k % 2],
                                out.at[pl.ds(base + pc * chunk, chunk)])
            copies.append(
                pltpu.async_copy(tab.at[idx.at[pl.ds(c * chunk, chunk)]],
                                 bufs[k % 2], sems[k % 2]))
        for k in (len(plan) - 2, len(plan) - 1):
            copies[k].wait()
            _, _, out, pc = plan[k]
            pltpu.sync_copy(bufs[k % 2],
                            out.at[pl.ds(base + pc * chunk, chunk)])

    return gather_k


def _mlp_body(u_ref, i_ref, uq_ref, iq_ref, w1u_ref, w1i_ref, b1_ref,
              w2_ref, b2_ref, o_ref):
    def select(g_ref, q_ref):
        q = q_ref[...]
        a = jnp.where(q < 2, g_ref[:, :64], g_ref[:, 64:])
        w = jnp.where((q & 1) == 0, a[:, :32], a[:, 32:])
        return _unpack_pair(w)

    u_sel = select(u_ref, uq_ref)
    i_sel = select(i_ref, iq_ref)
    dn = (((1,), (1,)), ((), ()))
    h = lax.dot_general(u_sel, w1u_ref[...], dn,
                        preferred_element_type=jnp.float32)
    h = h + lax.dot_general(i_sel, w1i_ref[...], dn,
                            preferred_element_type=jnp.float32)
    h = jnp.maximum(h + b1_ref[...], 0.0)
    o = lax.dot_general(h, w2_ref[...], dn, preferred_element_type=jnp.float32)
    o_ref[...] = o + b2_ref[...]


def kernel(user_ids, item_ids, user_table, item_table, W1, b1, W2, b2):
    batch = user_ids.shape[0]
    n_rows, emb = user_table.shape
    hidden = W1.shape[0]

    uids = user_ids.astype(jnp.int32)
    iids = item_ids.astype(jnp.int32)

    repack = _make_repack(n_rows, emb)
    u_packed = repack(user_table.T)
    i_packed = repack(item_table.T)

    def packed_idx(ids):
        return (ids // _BN) * _QN + (ids % _BN) % _QN

    gather_k = _make_gather(batch)
    u_slots, i_slots = gather_k(packed_idx(uids), packed_idx(iids),
                                u_packed, i_packed)

    u_q = ((uids % _BN) // _QN).astype(jnp.int32).reshape(batch, 1)
    i_q = ((iids % _BN) // _QN).astype(jnp.int32).reshape(batch, 1)
    w1u = W1[:, :emb].astype(jnp.bfloat16)
    w1i = W1[:, emb:].astype(jnp.bfloat16)
    b1r = b1.reshape(1, hidden)
    b2r = b2.reshape(1, hidden)

    bm = 2048
    grid = (batch // bm,)
    out = pl.pallas_call(
        _mlp_body,
        grid=grid,
        in_specs=[
            pl.BlockSpec((bm, 128), lambda i: (i, 0)),
            pl.BlockSpec((bm, 128), lambda i: (i, 0)),
            pl.BlockSpec((bm, 1), lambda i: (i, 0)),
            pl.BlockSpec((bm, 1), lambda i: (i, 0)),
            pl.BlockSpec((hidden, emb), lambda i: (0, 0)),
            pl.BlockSpec((hidden, emb), lambda i: (0, 0)),
            pl.BlockSpec((1, hidden), lambda i: (0, 0)),
            pl.BlockSpec((hidden, hidden), lambda i: (0, 0)),
            pl.BlockSpec((1, hidden), lambda i: (0, 0)),
        ],
        out_specs=pl.BlockSpec((bm, hidden), lambda i: (i, 0)),
        out_shape=jax.ShapeDtypeStruct((batch, hidden), jnp.float32),
    )(u_slots, i_slots, u_q, i_q, w1u, w1i, b1r, W2, b2r)
    return out


# speedup vs baseline: 1.9498x; 1.9498x over previous
import functools

import jax
import jax.numpy as jnp
from jax import lax
from jax.experimental import pallas as pl
from jax.experimental.pallas import tpu as pltpu
from jax.experimental.pallas import tpu_sc as plsc

_NUM_SC_CORES = 2
_NUM_SC_SUBCORES = 16
_NW = _NUM_SC_CORES * _NUM_SC_SUBCORES
_BN = 16384
_QN = _BN // 4


def _unpack_pair(w):
    lo = lax.convert_element_type(w & 0xFFFF, jnp.uint16)
    hi = lax.convert_element_type((w >> 16) & 0xFFFF, jnp.uint16)
    return jnp.concatenate(
        [lax.bitcast_convert_type(lo, jnp.bfloat16),
         lax.bitcast_convert_type(hi, jnp.bfloat16)], axis=1)


def _repack_one(t_ref, o_ref, emb):
    x = lax.bitcast_convert_type(t_ref[...], jnp.uint32)
    h = emb // 2
    quads = []
    for j in range(4):
        xq = x[:, j * _QN:(j + 1) * _QN]
        quads.append((xq[h:, :] & jnp.uint32(0xFFFF0000)) | (xq[:h, :] >> 16))
    w = jnp.concatenate(quads, axis=0)
    o_ref[...] = lax.bitcast_convert_type(w, jnp.int32).T


def _repack_body(tu_ref, ti_ref, ou_ref, oi_ref):
    emb = tu_ref.shape[0]
    _repack_one(tu_ref, ou_ref, emb)
    _repack_one(ti_ref, oi_ref, emb)


def _make_repack(n_rows: int, emb: int):
    n_blocks = (n_rows + _BN - 1) // _BN
    out_sd = jax.ShapeDtypeStruct((n_blocks * _QN, 128), jnp.int32)

    def repack(ut_t, it_t):
        return pl.pallas_call(
            _repack_body,
            grid=(n_blocks,),
            in_specs=[pl.BlockSpec((emb, _BN), lambda i: (0, i)),
                      pl.BlockSpec((emb, _BN), lambda i: (0, i))],
            out_specs=[pl.BlockSpec((_QN, 128), lambda i: (i, 0)),
                       pl.BlockSpec((_QN, 128), lambda i: (i, 0))],
            out_shape=[out_sd, out_sd],
        )(ut_t, it_t)

    return repack


def _make_gather(batch: int):
    b_per_w = batch // _NW
    chunk = 128
    n_chunks = b_per_w // chunk
    mesh = plsc.VectorSubcoreMesh(
        core_axis_name="c", subcore_axis_name="s",
        num_cores=_NUM_SC_CORES, num_subcores=_NUM_SC_SUBCORES)

    @functools.partial(
        pl.kernel,
        mesh=mesh,
        out_type=[
            jax.ShapeDtypeStruct((batch, 128), jnp.int32),
            jax.ShapeDtypeStruct((batch, 128), jnp.int32),
        ],
        scratch_types=[
            pltpu.VMEM((b_per_w,), jnp.int32),
            pltpu.VMEM((b_per_w,), jnp.int32),
            pltpu.VMEM((chunk, 128), jnp.int32),
            pltpu.VMEM((chunk, 128), jnp.int32),
            pltpu.SemaphoreType.DMA,
            pltpu.SemaphoreType.DMA,
        ],
    )
    def gather_k(uids_hbm, iids_hbm, utab_hbm, itab_hbm, uout_hbm, iout_hbm,
                 uidx_v, iidx_v, buf0, buf1, sem0, sem1):
        wid = lax.axis_index("s") * _NUM_SC_CORES + lax.axis_index("c")
        base = wid * b_per_w
        pltpu.sync_copy(uids_hbm.at[pl.ds(base, b_per_w)], uidx_v)
        pltpu.sync_copy(iids_hbm.at[pl.ds(base, b_per_w)], iidx_v)

        plan = [(utab_hbm, uidx_v, uout_hbm, c) for c in range(n_chunks)]
        plan += [(itab_hbm, iidx_v, iout_hbm, c) for c in range(n_chunks)]
        bufs = [buf0, buf1]
        sems = [sem0, sem1]
        copies = []
        for k, (tab, idx, _, c) in enumerate(plan):
            if k >= 2:
                copies[k - 2].wait()
                _, _, out, pc = plan[k - 2]
                pltpu.sync_copy(bufs[k % 2],
                                out.at[pl.ds(base + pc * chunk, chunk)])
            copies.append(
                pltpu.async_copy(tab.at[idx.at[pl.ds(c * chunk, chunk)]],
                                 bufs[k % 2], sems[k % 2]))
        for k in (len(plan) - 2, len(plan) - 1):
            copies[k].wait()
            _, _, out, pc = plan[k]
            pltpu.sync_copy(bufs[k % 2],
                            out.at[pl.ds(base + pc * chunk, chunk)])

    return gather_k


def _mlp_body(u_ref, i_ref, uq_ref, iq_ref, w1u_ref, w1i_ref, b1_ref,
              w2_ref, b2_ref, o_ref):
    def select(g_ref, q_ref):
        q = q_ref[...]
        a = jnp.where(q < 2, g_ref[:, :64], g_ref[:, 64:])
        w = jnp.where((q & 1) == 0, a[:, :32], a[:, 32:])
        return _unpack_pair(w)

    u_sel = select(u_ref, uq_ref)
    i_sel = select(i_ref, iq_ref)
    dn = (((1,), (1,)), ((), ()))
    h = lax.dot_general(u_sel, w1u_ref[...], dn,
                        preferred_element_type=jnp.float32)
    h = h + lax.dot_general(i_sel, w1i_ref[...], dn,
                            preferred_element_type=jnp.float32)
    h = jnp.maximum(h + b1_ref[...], 0.0)
    o = lax.dot_general(h, w2_ref[...], dn, preferred_element_type=jnp.float32)
    o_ref[...] = o + b2_ref[...]


def kernel(user_ids, item_ids, user_table, item_table, W1, b1, W2, b2):
    batch = user_ids.shape[0]
    n_rows, emb = user_table.shape
    hidden = W1.shape[0]

    uids = user_ids.astype(jnp.int32)
    iids = item_ids.astype(jnp.int32)

    repack = _make_repack(n_rows, emb)
    u_packed, i_packed = repack(user_table.T, item_table.T)

    def packed_idx(ids):
        return (ids // _BN) * _QN + (ids % _BN) % _QN

    gather_k = _make_gather(batch)
    u_slots, i_slots = gather_k(packed_idx(uids), packed_idx(iids),
                                u_packed, i_packed)

    u_q = ((uids % _BN) // _QN).astype(jnp.int32).reshape(batch, 1)
    i_q = ((iids % _BN) // _QN).astype(jnp.int32).reshape(batch, 1)
    w1u = W1[:, :emb].astype(jnp.bfloat16)
    w1i = W1[:, emb:].astype(jnp.bfloat16)
    b1r = b1.reshape(1, hidden)
    b2r = b2.reshape(1, hidden)

    bm = 2048
    grid = (batch // bm,)
    out = pl.pallas_call(
        _mlp_body,
        grid=grid,
        in_specs=[
            pl.BlockSpec((bm, 128), lambda i: (i, 0)),
            pl.BlockSpec((bm, 128), lambda i: (i, 0)),
            pl.BlockSpec((bm, 1), lambda i: (i, 0)),
            pl.BlockSpec((bm, 1), lambda i: (i, 0)),
            pl.BlockSpec((hidden, emb), lambda i: (0, 0)),
            pl.BlockSpec((hidden, emb), lambda i: (0, 0)),
            pl.BlockSpec((1, hidden), lambda i: (0, 0)),
            pl.BlockSpec((hidden, hidden), lambda i: (0, 0)),
            pl.BlockSpec((1, hidden), lambda i: (0, 0)),
        ],
        out_specs=pl.BlockSpec((bm, hidden), lambda i: (i, 0)),
        out_shape=jax.ShapeDtypeStruct((batch, hidden), jnp.float32),
    )(u_slots, i_slots, u_q, i_q, w1u, w1i, b1r, W2, b2r)
    return out
